# Initial kernel scaffold; baseline (speedup 1.0000x reference)
import jax
import jax.numpy as jnp
from jax.experimental import pallas as pl


def kernel(values, labels, emb_table, W, b):
    raise NotImplementedError("write your pallas kernel here")


# speedup vs baseline: 3.0225x; 3.0225x over previous
import functools

import jax
import jax.numpy as jnp
from jax import lax
from jax.experimental import pallas as pl
from jax.experimental.pallas import tpu as pltpu
from jax.experimental.pallas import tpu_sc as plsc

NC, NS = 2, 16
NW = NC * NS
CHUNK = 128


@functools.partial(jax.jit, static_argnums=(2, 3, 4))
def _sc_gather(labels3, emb_wide_table, n_chunks, n_rows, emb_d):
    mesh = plsc.VectorSubcoreMesh(
        core_axis_name="c", subcore_axis_name="s",
        num_cores=NC, num_subcores=NS,
    )

    @functools.partial(
        pl.kernel,
        out_type=jax.ShapeDtypeStruct((NW * n_chunks * CHUNK, emb_d), jnp.float32),
        mesh=mesh,
        scratch_types=[
            pltpu.VMEM((n_chunks, CHUNK), jnp.int32),
            pltpu.VMEM((CHUNK, emb_d), jnp.float32),
            pltpu.SemaphoreType.DMA,
        ],
        compiler_params=pltpu.CompilerParams(use_tc_tiling_on_sc=False),
    )
    def gather_kernel(labels_hbm, table2d, out_hbm, idx_v, rows_v, sem):
        wid = lax.axis_index("c") * NS + lax.axis_index("s")
        base = wid * (n_chunks * CHUNK)
        pltpu.sync_copy(labels_hbm.at[wid], idx_v)

        def body(j, carry):
            pltpu.async_copy(table2d.at[idx_v.at[j]], rows_v, sem).wait()
            pltpu.sync_copy(rows_v, out_hbm.at[pl.ds(base + j * CHUNK, CHUNK)])
            return carry

        lax.fori_loop(0, n_chunks, body, 0)

    return gather_kernel(labels3, emb_wide_table)


def _mm_body(v_ref, e_ref, wv_ref, we_ref, b_ref, o_ref):
    acc = jnp.dot(v_ref[...], wv_ref[...], preferred_element_type=jnp.float32)
    acc += jnp.dot(e_ref[...], we_ref[...], preferred_element_type=jnp.float32)
    o_ref[...] = acc + b_ref[...]


@jax.jit
def _tc_project(values_flat, emb_flat, Wv, We, b2):
    N, VD = values_flat.shape
    ED = emb_flat.shape[1]
    LD = Wv.shape[1]
    R = 2048
    grid = (N // R,)
    return pl.pallas_call(
        _mm_body,
        grid=grid,
        in_specs=[
            pl.BlockSpec((R, VD), lambda i: (i, 0)),
            pl.BlockSpec((R, ED), lambda i: (i, 0)),
            pl.BlockSpec((VD, LD), lambda i: (0, 0)),
            pl.BlockSpec((ED, LD), lambda i: (0, 0)),
            pl.BlockSpec((1, LD), lambda i: (0, 0)),
        ],
        out_specs=pl.BlockSpec((R, LD), lambda i: (i, 0)),
        out_shape=jax.ShapeDtypeStruct((N, LD), jnp.float32),
    )(values_flat, emb_flat, Wv, We, b2)


def kernel(values, labels, emb_table, W, b):
    B, T, VD = values.shape
    ED = emb_table.shape[1]
    LD = W.shape[1]
    N = B * T

    n_chunks = N // (NW * CHUNK)
    labels3 = labels.reshape(NW, n_chunks, CHUNK).astype(jnp.int32)
    emb_flat = _sc_gather(labels3, emb_table, n_chunks, emb_table.shape[0], ED)

    values_flat = values.reshape(N, VD)
    Wv = W[:VD]
    We = W[VD:VD + ED]
    b2 = b.reshape(1, LD)
    out = _tc_project(values_flat, emb_flat, Wv, We, b2)
    return out.reshape(B, T, LD)


# speedup vs baseline: 3.4789x; 1.1510x over previous
import functools

import jax
import jax.numpy as jnp
from jax import lax
from jax.experimental import pallas as pl
from jax.experimental.pallas import tpu as pltpu
from jax.experimental.pallas import tpu_sc as plsc

NC, NS = 2, 16
NW = NC * NS
CHUNK = 128


@functools.partial(jax.jit, static_argnums=(2, 3, 4))
def _sc_gather(labels3, emb_wide_table, n_chunks, n_rows, emb_d):
    mesh = plsc.VectorSubcoreMesh(
        core_axis_name="c", subcore_axis_name="s",
        num_cores=NC, num_subcores=NS,
    )

    L = 8
    G = 4
    assert n_chunks > L

    @functools.partial(
        pl.kernel,
        out_type=jax.ShapeDtypeStruct((NW * n_chunks * CHUNK, emb_d), jnp.float32),
        mesh=mesh,
        scratch_types=[
            pltpu.VMEM((n_chunks, CHUNK), jnp.int32),
            pltpu.VMEM((L, CHUNK, emb_d), jnp.float32),
            pltpu.SemaphoreType.DMA((L,)),
            pltpu.SemaphoreType.DMA((L,)),
        ],
        compiler_params=pltpu.CompilerParams(use_tc_tiling_on_sc=False),
    )
    def gather_kernel(labels_hbm, table2d, out_hbm, idx_v, rows_v, gsem, wsem):
        wid = lax.axis_index("c") * NS + lax.axis_index("s")
        base = wid * (n_chunks * CHUNK)
        pltpu.sync_copy(labels_hbm.at[wid], idx_v)

        for k in range(G):
            pltpu.async_copy(
                table2d.at[idx_v.at[k]], rows_v.at[k % L], gsem.at[k % L])

        def body(j, carry):
            slot = lax.rem(j, L)
            pltpu.make_async_copy(
                table2d.at[idx_v.at[j]], rows_v.at[slot], gsem.at[slot]).wait()
            pltpu.async_copy(
                rows_v.at[slot],
                out_hbm.at[pl.ds(base + j * CHUNK, CHUNK)],
                wsem.at[slot])

            nslot = lax.rem(j + G, L)

            @pl.when(j + G < n_chunks)
            def _issue_next():
                @pl.when(j + G >= L)
                def _drain_write():
                    pltpu.make_async_copy(
                        rows_v.at[nslot],
                        out_hbm.at[pl.ds(base, CHUNK)],
                        wsem.at[nslot]).wait()

                pltpu.async_copy(
                    table2d.at[idx_v.at[j + G]], rows_v.at[nslot],
                    gsem.at[nslot])

            return carry

        lax.fori_loop(0, n_chunks, body, 0)

        for j in range(max(0, n_chunks + G - L), n_chunks):
            pltpu.make_async_copy(
                rows_v.at[j % L],
                out_hbm.at[pl.ds(base, CHUNK)],
                wsem.at[j % L]).wait()

    return gather_kernel(labels3, emb_wide_table)


def _mm_body(v_ref, e_ref, wv_ref, we_ref, b_ref, o_ref):
    acc = jnp.dot(v_ref[...], wv_ref[...], preferred_element_type=jnp.float32)
    acc += jnp.dot(e_ref[...], we_ref[...], preferred_element_type=jnp.float32)
    o_ref[...] = acc + b_ref[...]


@jax.jit
def _tc_project(values_flat, emb_flat, Wv, We, b2):
    N, VD = values_flat.shape
    ED = emb_flat.shape[1]
    LD = Wv.shape[1]
    R = 2048
    grid = (N // R,)
    return pl.pallas_call(
        _mm_body,
        grid=grid,
        in_specs=[
            pl.BlockSpec((R, VD), lambda i: (i, 0)),
            pl.BlockSpec((R, ED), lambda i: (i, 0)),
            pl.BlockSpec((VD, LD), lambda i: (0, 0)),
            pl.BlockSpec((ED, LD), lambda i: (0, 0)),
            pl.BlockSpec((1, LD), lambda i: (0, 0)),
        ],
        out_specs=pl.BlockSpec((R, LD), lambda i: (i, 0)),
        out_shape=jax.ShapeDtypeStruct((N, LD), jnp.float32),
    )(values_flat, emb_flat, Wv, We, b2)


def kernel(values, labels, emb_table, W, b):
    B, T, VD = values.shape
    ED = emb_table.shape[1]
    LD = W.shape[1]
    N = B * T

    n_chunks = N // (NW * CHUNK)
    labels3 = labels.reshape(NW, n_chunks, CHUNK).astype(jnp.int32)
    emb_flat = _sc_gather(labels3, emb_table, n_chunks, emb_table.shape[0], ED)

    values_flat = values.reshape(N, VD)
    Wv = W[:VD]
    We = W[VD:VD + ED]
    b2 = b.reshape(1, LD)
    out = _tc_project(values_flat, emb_flat, Wv, We, b2)
    return out.reshape(B, T, LD)


# speedup vs baseline: 3.5143x; 1.0102x over previous
import functools

import jax
import jax.numpy as jnp
from jax import lax
from jax.experimental import pallas as pl
from jax.experimental.pallas import tpu as pltpu
from jax.experimental.pallas import tpu_sc as plsc

NC, NS = 2, 16
NW = NC * NS
CHUNK = 128
NSLICES = 4
ROWS_BLK = 2048


@functools.partial(jax.jit, static_argnums=(2, 3, 4))
def _sc_gather(labels3, emb_table, n_chunks, n_rows, emb_d):
    mesh = plsc.VectorSubcoreMesh(
        core_axis_name="c", subcore_axis_name="s",
        num_cores=NC, num_subcores=NS,
    )
    L = 8
    G = 4
    assert n_chunks > L

    @functools.partial(
        pl.kernel,
        out_type=jax.ShapeDtypeStruct((NW * n_chunks * CHUNK, emb_d), jnp.float32),
        mesh=mesh,
        scratch_types=[
            pltpu.VMEM((n_chunks, CHUNK), jnp.int32),
            pltpu.VMEM((L, CHUNK, emb_d), jnp.float32),
            pltpu.SemaphoreType.DMA((L,)),
            pltpu.SemaphoreType.DMA((L,)),
        ],
        compiler_params=pltpu.CompilerParams(use_tc_tiling_on_sc=False),
    )
    def gather_kernel(labels_hbm, table2d, out_hbm, idx_v, rows_v, gsem, wsem):
        wid = lax.axis_index("c") * NS + lax.axis_index("s")
        base = wid * (n_chunks * CHUNK)
        pltpu.sync_copy(labels_hbm.at[wid], idx_v)

        for k in range(G):
            pltpu.async_copy(
                table2d.at[idx_v.at[k]], rows_v.at[k % L], gsem.at[k % L])

        def body(j, carry):
            slot = lax.rem(j, L)
            pltpu.make_async_copy(
                table2d.at[idx_v.at[j]], rows_v.at[slot], gsem.at[slot]).wait()
            pltpu.async_copy(
                rows_v.at[slot],
                out_hbm.at[pl.ds(base + j * CHUNK, CHUNK)],
                wsem.at[slot])

            nslot = lax.rem(j + G, L)

            @pl.when(j + G < n_chunks)
            def _issue_next():
                @pl.when(j + G >= L)
                def _drain_write():
                    pltpu.make_async_copy(
                        rows_v.at[nslot],
                        out_hbm.at[pl.ds(base, CHUNK)],
                        wsem.at[nslot]).wait()

                pltpu.async_copy(
                    table2d.at[idx_v.at[j + G]], rows_v.at[nslot],
                    gsem.at[nslot])

            return carry

        lax.fori_loop(0, n_chunks, body, 0)

        for j in range(max(0, n_chunks + G - L), n_chunks):
            pltpu.make_async_copy(
                rows_v.at[j % L],
                out_hbm.at[pl.ds(base, CHUNK)],
                wsem.at[j % L]).wait()

    return gather_kernel(labels3, emb_table)


def _mm_first_body(v_ref, e_ref, wv_ref, we_ref, b_ref, o_ref):
    acc = jnp.dot(v_ref[...], wv_ref[...], preferred_element_type=jnp.float32)
    acc += jnp.dot(e_ref[...], we_ref[...], preferred_element_type=jnp.float32)
    o_ref[...] = acc + b_ref[...]


def _mm_chain_body(prev_ref, v_ref, e_ref, wv_ref, we_ref, b_ref, o_ref):
    del prev_ref
    _mm_first_body(v_ref, e_ref, wv_ref, we_ref, b_ref, o_ref)


@functools.partial(jax.jit, static_argnums=(6, 7))
def _tc_project_slice(prev, values_flat, emb_s, Wv, We, b2, s, n_total):
    Q, ED = emb_s.shape
    VD = values_flat.shape[1]
    LD = Wv.shape[1]
    R = ROWS_BLK
    blk_off = s * (Q // R)
    grid = (Q // R,)

    common_in_specs = [
        pl.BlockSpec((R, VD), lambda i: (blk_off + i, 0)),
        pl.BlockSpec((R, ED), lambda i: (i, 0)),
        pl.BlockSpec((VD, LD), lambda i: (0, 0)),
        pl.BlockSpec((ED, LD), lambda i: (0, 0)),
        pl.BlockSpec((1, LD), lambda i: (0, 0)),
    ]
    out_spec = pl.BlockSpec((R, LD), lambda i: (blk_off + i, 0))
    out_shape = jax.ShapeDtypeStruct((n_total, LD), jnp.float32)

    if prev is None:
        return pl.pallas_call(
            _mm_first_body,
            grid=grid,
            in_specs=common_in_specs,
            out_specs=out_spec,
            out_shape=out_shape,
        )(values_flat, emb_s, Wv, We, b2)
    return pl.pallas_call(
        _mm_chain_body,
        grid=grid,
        in_specs=[pl.BlockSpec(memory_space=pl.ANY)] + common_in_specs,
        out_specs=out_spec,
        out_shape=out_shape,
        input_output_aliases={0: 0},
    )(prev, values_flat, emb_s, Wv, We, b2)


def kernel(values, labels, emb_table, W, b):
    B, T, VD = values.shape
    ED = emb_table.shape[1]
    LD = W.shape[1]
    N = B * T
    Q = N // NSLICES

    labels_flat = labels.reshape(N).astype(jnp.int32)
    values_flat = values.reshape(N, VD)
    Wv = W[:VD]
    We = W[VD:VD + ED]
    b2 = b.reshape(1, LD)

    n_chunks = Q // (NW * CHUNK)
    embs = []
    for s in range(NSLICES):
        labels3 = lax.dynamic_slice_in_dim(labels_flat, s * Q, Q).reshape(
            NW, n_chunks, CHUNK)
        embs.append(_sc_gather(labels3, emb_table, n_chunks,
                               emb_table.shape[0], ED))

    out = None
    for s in range(NSLICES):
        out = _tc_project_slice(out, values_flat, embs[s], Wv, We, b2, s, N)
    return out.reshape(B, T, LD)
